# baseline (speedup 1.0000x reference)
import functools

import jax
import jax.numpy as jnp
from jax import lax
from jax.experimental import pallas as pl
from jax.experimental.pallas import tpu as pltpu
from jax.experimental.pallas import tpu_sc as plsc

B = 1024
D = 32
V = 100000
TV = 2048

_info = plsc.get_sparse_core_info()
_NC, _NS = _info.num_cores, _info.num_subcores
_NW = _NC * _NS
_BPW = B // _NW

_sc_mesh = plsc.VectorSubcoreMesh(core_axis_name="c", subcore_axis_name="s")


@functools.partial(
    pl.kernel,
    mesh=_sc_mesh,
    out_type=jax.ShapeDtypeStruct((B, D), jnp.float32),
    scratch_types=[
        pltpu.VMEM((_BPW,), jnp.int32),
        pltpu.VMEM((_BPW, D), jnp.float32),
        pltpu.SemaphoreType.DMA,
    ],
    compiler_params=pltpu.CompilerParams(use_tc_tiling_on_sc=False),
)
def _sc_gather(idx_hbm, table_hbm, out_hbm, idx_v, rows_v, sem):
    wid = lax.axis_index("s") * _NC + lax.axis_index("c")
    base = wid * _BPW
    pltpu.sync_copy(idx_hbm.at[pl.ds(base, _BPW)], idx_v)
    pltpu.async_copy(table_hbm.at[idx_v], rows_v, sem).wait()
    pltpu.sync_copy(rows_v, out_hbm.at[pl.ds(base, _BPW)])


def _mm_body(rows_ref, w_ref, b_ref, out_ref, membT_ref):
    @pl.when(pl.program_id(0) == 0)
    def _():
        membT_ref[0:D, :] = jnp.transpose(rows_ref[...])
        membT_ref[D : D + 1, :] = jnp.ones((1, B), jnp.float32)

    w_aug = jnp.concatenate([w_ref[...], b_ref[...]], axis=0)
    out_ref[...] = lax.dot_general(
        w_aug,
        membT_ref[...],
        (((0,), (0,)), ((), ())),
        preferred_element_type=jnp.float32,
    )


def _project(rows, W, b2d):
    n_tiles = pl.cdiv(V, TV)
    return pl.pallas_call(
        _mm_body,
        grid=(n_tiles,),
        in_specs=[
            pl.BlockSpec((B, D), lambda i: (0, 0)),
            pl.BlockSpec((D, TV), lambda i: (0, i)),
            pl.BlockSpec((1, TV), lambda i: (0, i)),
        ],
        out_specs=pl.BlockSpec((TV, B), lambda i: (i, 0)),
        out_shape=jax.ShapeDtypeStruct((V, B), jnp.float32),
        scratch_shapes=[pltpu.VMEM((D + 1, B), jnp.float32)],
    )(rows, W, b2d)


def kernel(input_tokens, emb_table, W, b):
    idx = input_tokens.reshape(-1).astype(jnp.int32)
    rows = _sc_gather(idx, emb_table)
    logitsT = _project(rows, W, b.reshape(1, V))
    return logitsT.T.reshape(B, 1, V)
